# baseline (speedup 1.0000x reference)
import functools

import jax
import jax.numpy as jnp
from jax import lax
from jax.experimental import pallas as pl
from jax.experimental.pallas import tpu as pltpu
from jax.experimental.pallas import tpu_sc as plsc

_B = 4
_S = 4096
_F = 1024
_S_SC = 1024
_NC = 2
_NS = 16
_NW = _NC * _NS
_S_PER_W = _S_SC // _NW
_CHUNK = 8
_NG = _S_PER_W // _CHUNK
_NP = _NG // 2
_RING = 3
_JV = _F // 16

_TC_BLK = 512


def _sc_body(x_hbm, pos_hbm, out_hbm, *scratch):
    xbufs = scratch[:_RING]
    pbufs = scratch[_RING:_RING + 2]
    in_sems = scratch[_RING + 2:_RING + 5]
    out_sems = scratch[_RING + 5:_RING + 8]
    pos_sems = scratch[_RING + 8:_RING + 10]
    wid = lax.axis_index("s") * _NC + lax.axis_index("c")
    s0 = wid * _S_PER_W

    def issue_in(g):
        r = g % _RING
        row0 = s0 + g * _CHUNK
        return pltpu.async_copy(
            x_hbm.at[:, pl.ds(row0, _CHUNK), :], xbufs[r], in_sems[r])

    def issue_pos(k):
        row0 = s0 + k * 2 * _CHUNK
        return pltpu.async_copy(
            pos_hbm.at[pl.ds(row0, 2 * _CHUNK), :], pbufs[k % 2], pos_sems[k % 2])

    in_h = [None] * _RING
    out_h = [None] * _RING
    pos_h = [None, None]
    pos_h[0] = issue_pos(0)
    if _NP > 1:
        pos_h[1] = issue_pos(1)
    in_h[0] = issue_in(0)
    in_h[1] = issue_in(1)

    for g in range(_NG):
        r = g % _RING
        k = g // 2
        gn = g + 2
        if gn < _NG:
            rn = gn % _RING
            if out_h[rn] is not None:
                out_h[rn].wait()
            in_h[rn] = issue_in(gn)
        if g % 2 == 0:
            pos_h[k % 2].wait()
        in_h[r].wait()
        xb = xbufs[r]
        pvb = pbufs[k % 2]
        roff = (g % 2) * _CHUNK

        def row_body(i, carry, xb=xb, pvb=pvb, roff=roff):

            def col_body(j, carry2, i=i, xb=xb, pvb=pvb, roff=roff):
                sl = pl.ds(j * 16, 16)
                p = pvb[roff + i, sl]
                for b in range(_B):
                    xb[b, i, sl] = xb[b, i, sl] + p
                return carry2

            return lax.fori_loop(0, _JV, col_body, carry, unroll=8)

        lax.fori_loop(0, _CHUNK, row_body, None)
        row0 = s0 + g * _CHUNK
        out_h[r] = pltpu.async_copy(
            xb, out_hbm.at[:, pl.ds(row0, _CHUNK), :], out_sems[r])
        if g % 2 == 1 and k + 2 < _NP:
            pos_h[k % 2] = issue_pos(k + 2)

    for h in out_h:
        if h is not None:
            h.wait()


def _sc_part(x, pos_table):
    mesh = plsc.VectorSubcoreMesh(core_axis_name="c", subcore_axis_name="s")
    scratch = []
    for _ in range(_RING):
        scratch.append(pltpu.VMEM((_B, _CHUNK, _F), jnp.float32))
    for _ in range(2):
        scratch.append(pltpu.VMEM((2 * _CHUNK, _F), jnp.float32))
    for _ in range(8):
        scratch.append(pltpu.SemaphoreType.DMA)
    run = functools.partial(
        pl.kernel,
        mesh=mesh,
        out_type=jax.ShapeDtypeStruct((_B, _S_SC, _F), jnp.float32),
        scratch_types=scratch,
        compiler_params=pltpu.CompilerParams(use_tc_tiling_on_sc=True),
    )(_sc_body)
    return run(x, pos_table)


def _tc_body(x_ref, p_ref, o_ref):
    o_ref[...] = x_ref[...] + p_ref[...][None, :, :]


def _tc_part(x, pos_table):
    nblk = (_S - _S_SC) // _TC_BLK
    off = _S_SC // _TC_BLK
    return pl.pallas_call(
        _tc_body,
        grid=(nblk,),
        in_specs=[
            pl.BlockSpec((_B, _TC_BLK, _F), lambda i: (0, i + off, 0)),
            pl.BlockSpec((_TC_BLK, _F), lambda i: (i + off, 0)),
        ],
        out_specs=pl.BlockSpec((_B, _TC_BLK, _F), lambda i: (0, i + off, 0)),
        out_shape=jax.ShapeDtypeStruct((_B, _S, _F), jnp.float32),
    )(x, pos_table)


def kernel(x, pos_table):
    sc_out = _sc_part(x, pos_table)
    tc_out = _tc_part(x, pos_table)
    return lax.dynamic_update_slice(tc_out, sc_out, (0, 0, 0))


# speedup vs baseline: 1.0042x; 1.0042x over previous
import functools

import jax
import jax.numpy as jnp
from jax import lax
from jax.experimental import pallas as pl
from jax.experimental.pallas import tpu as pltpu
from jax.experimental.pallas import tpu_sc as plsc

_B = 4
_S = 4096
_F = 1024
_NC = 2
_NS = 16
_NW = _NC * _NS
_S_PER_W = _S // _NW
_CHUNK = 8
_NG = _S_PER_W // _CHUNK
_NP = _NG // 2
_RING = 3
_JV = _F // 16


def _sc_body(x_hbm, pos_hbm, out_hbm, *scratch):
    xbufs = scratch[:_RING]
    pbufs = scratch[_RING:_RING + 2]
    in_sems = scratch[_RING + 2:_RING + 5]
    out_sems = scratch[_RING + 5:_RING + 8]
    pos_sems = scratch[_RING + 8:_RING + 10]
    wid = lax.axis_index("s") * _NC + lax.axis_index("c")
    s0 = wid * _S_PER_W

    def issue_in(g):
        r = g % _RING
        row0 = s0 + g * _CHUNK
        return pltpu.async_copy(
            x_hbm.at[:, pl.ds(row0, _CHUNK), :], xbufs[r], in_sems[r])

    def issue_pos(k):
        row0 = s0 + k * 2 * _CHUNK
        return pltpu.async_copy(
            pos_hbm.at[pl.ds(row0, 2 * _CHUNK), :], pbufs[k % 2], pos_sems[k % 2])

    in_h = [None] * _RING
    out_h = [None] * _RING
    pos_h = [None, None]
    pos_h[0] = issue_pos(0)
    pos_h[1] = issue_pos(1)
    in_h[0] = issue_in(0)
    in_h[1] = issue_in(1)

    for g in range(_NG):
        r = g % _RING
        k = g // 2
        gn = g + 2
        if gn < _NG:
            rn = gn % _RING
            if out_h[rn] is not None:
                out_h[rn].wait()
            in_h[rn] = issue_in(gn)
        if g % 2 == 0:
            pos_h[k % 2].wait()
        in_h[r].wait()
        xb = xbufs[r]
        pvb = pbufs[k % 2]
        roff = (g % 2) * _CHUNK

        def row_body(i, carry, xb=xb, pvb=pvb, roff=roff):

            def col_body(j, carry2, i=i, xb=xb, pvb=pvb, roff=roff):
                sl = pl.ds(j * 16, 16)
                p = pvb[roff + i, sl]
                for b in range(_B):
                    xb[b, i, sl] = xb[b, i, sl] + p
                return carry2

            return lax.fori_loop(0, _JV, col_body, carry, unroll=8)

        lax.fori_loop(0, _CHUNK, row_body, None)
        row0 = s0 + g * _CHUNK
        out_h[r] = pltpu.async_copy(
            xb, out_hbm.at[:, pl.ds(row0, _CHUNK), :], out_sems[r])
        if g % 2 == 1 and k + 2 < _NP:
            pos_h[k % 2] = issue_pos(k + 2)

    for h in out_h:
        if h is not None:
            h.wait()


def kernel(x, pos_table):
    B, S, F = x.shape
    mesh = plsc.VectorSubcoreMesh(core_axis_name="c", subcore_axis_name="s")
    scratch = []
    for _ in range(_RING):
        scratch.append(pltpu.VMEM((_B, _CHUNK, _F), jnp.float32))
    for _ in range(2):
        scratch.append(pltpu.VMEM((2 * _CHUNK, _F), jnp.float32))
    for _ in range(8):
        scratch.append(pltpu.SemaphoreType.DMA)
    run = functools.partial(
        pl.kernel,
        mesh=mesh,
        out_type=jax.ShapeDtypeStruct((B, S, F), jnp.float32),
        scratch_types=scratch,
        compiler_params=pltpu.CompilerParams(use_tc_tiling_on_sc=True),
    )(_sc_body)
    return run(x, pos_table)
